# baseline (speedup 1.0000x reference)
import functools

import jax
import jax.numpy as jnp
from jax.experimental import pallas as pl
from jax.experimental.pallas import tpu as pltpu

TN1 = 256
TN2 = 512


def _l1_kernel(B, TN, lower_ref, xT_ref, Wtop_ref, Wbot_ref, b1t_ref,
               h_ref, L16_ref):
    i = pl.program_id(0)
    L16 = lower_ref[...].astype(jnp.bfloat16)
    L16_ref[...] = L16
    xT16 = xT_ref[...].astype(jnp.bfloat16)
    s = jnp.dot(L16, xT16, preferred_element_type=jnp.float32)
    xr = xT_ref[pl.ds(i * TN, TN), :]
    hb = (jnp.dot(xr, Wtop_ref[...], preferred_element_type=jnp.float32)
          + jnp.dot(s, Wbot_ref[...], preferred_element_type=jnp.float32)
          + b1t_ref[...])
    h_ref[...] = jnp.maximum(hb, 0.0).astype(jnp.bfloat16)


def _l2_kernel(B, HD, TN, L16_ref, h_ref, BW20_ref, BW21_ref, b2t_ref,
               We_ref, be_ref, Wo_ref, bo_ref, out_ref, m_ref):
    j = pl.program_id(0)
    nsteps = pl.num_programs(0)
    Lb = L16_ref[...]
    agg16 = jnp.dot(Lb, h_ref[...],
                    preferred_element_type=jnp.float32
                    ).astype(jnp.bfloat16)
    Hi = h_ref[pl.ds(j * TN, TN), :]
    G = (jnp.dot(Hi, BW20_ref[...], preferred_element_type=jnp.float32)
         + jnp.dot(agg16, BW21_ref[...], preferred_element_type=jnp.float32)
         + b2t_ref[...])
    G = jnp.maximum(G, 0.0)
    Gm = jnp.max(G, axis=0, keepdims=True)

    @pl.when(j == 0)
    def _():
        m_ref[...] = Gm

    @pl.when(j > 0)
    def _():
        m_ref[...] = jnp.maximum(m_ref[...], Gm)

    @pl.when(j == nsteps - 1)
    def _():
        mm = m_ref[...].reshape(B, HD)
        t = jnp.dot(mm, We_ref[...], preferred_element_type=jnp.float32)
        t = jnp.maximum(t + be_ref[...], 0.0)
        out_ref[...] = (jnp.dot(t, Wo_ref[...],
                                preferred_element_type=jnp.float32)
                        + bo_ref[...])


def kernel(x, lower, _, W1_0, W1_1, b1, W2_0, W2_1, b2, We, be, Wo, bo):
    B, N, _d = x.shape
    HD = W1_0.shape[1]
    HFF = We.shape[1]
    NC = Wo.shape[1]

    xT = x[:, :, 0].T
    eyeB = jnp.eye(B, dtype=jnp.float32)
    Wtop = jnp.kron(eyeB, W1_0)
    Wbot = jnp.kron(eyeB, W1_1)
    b1t = jnp.tile(b1, B).reshape(1, B * HD)
    BW20 = jnp.kron(eyeB, W2_0).astype(jnp.bfloat16)
    BW21 = jnp.kron(eyeB, W2_1).astype(jnp.bfloat16)
    b2t = jnp.tile(b2, B).reshape(1, B * HD)
    ber = be.reshape(1, HFF)
    bor = bo.reshape(1, NC)

    cidx = lambda i: (0, 0)
    h2d, L16 = pl.pallas_call(
        functools.partial(_l1_kernel, B, TN1),
        grid=(N // TN1,),
        in_specs=[
            pl.BlockSpec((TN1, N), lambda i: (i, 0)),
            pl.BlockSpec((N, B), cidx),
            pl.BlockSpec((B, B * HD), cidx),
            pl.BlockSpec((B, B * HD), cidx),
            pl.BlockSpec((1, B * HD), cidx),
        ],
        out_specs=[
            pl.BlockSpec((TN1, B * HD), lambda i: (i, 0)),
            pl.BlockSpec((TN1, N), lambda i: (i, 0)),
        ],
        out_shape=[
            jax.ShapeDtypeStruct((N, B * HD), jnp.bfloat16),
            jax.ShapeDtypeStruct((N, N), jnp.bfloat16),
        ],
    )(lower, xT, Wtop, Wbot, b1t)

    out = pl.pallas_call(
        functools.partial(_l2_kernel, B, HD, TN2),
        grid=(N // TN2,),
        in_specs=[
            pl.BlockSpec((TN2, N), lambda i: (i, 0)),
            pl.BlockSpec((N, B * HD), cidx),
            pl.BlockSpec((B * HD, B * HD), cidx),
            pl.BlockSpec((B * HD, B * HD), cidx),
            pl.BlockSpec((1, B * HD), cidx),
            pl.BlockSpec((HD, HFF), cidx),
            pl.BlockSpec((1, HFF), cidx),
            pl.BlockSpec((HFF, NC), cidx),
            pl.BlockSpec((1, NC), cidx),
        ],
        out_specs=pl.BlockSpec((B, NC), cidx),
        out_shape=jax.ShapeDtypeStruct((B, NC), jnp.float32),
        scratch_shapes=[pltpu.VMEM((1, B * HD), jnp.float32)],
    )(L16, h2d, BW20, BW21, b2t, We, ber, Wo, bor)

    return out


# speedup vs baseline: 1.3571x; 1.3571x over previous
import functools

import jax
import jax.numpy as jnp
from jax.experimental import pallas as pl
from jax.experimental.pallas import tpu as pltpu

TN1 = 256
TN2 = 512


def _l1_kernel(B, TN, lower_ref, xT_ref, W10_ref, W11_ref, b1_ref, h_ref):
    i = pl.program_id(0)
    L = lower_ref[...]
    xT = xT_ref[...]
    s = jnp.dot(L, xT, preferred_element_type=jnp.float32)
    xr = xT_ref[pl.ds(i * TN, TN), :]
    W10 = W10_ref[...]
    W11 = W11_ref[...]
    b1 = b1_ref[...]
    pieces = []
    for b in range(B):
        hb = xr[:, b:b + 1] * W10 + s[:, b:b + 1] * W11 + b1
        pieces.append(jnp.maximum(hb, 0.0))
    h_ref[...] = jnp.concatenate(pieces, axis=1).astype(jnp.bfloat16)


def _l2_kernel(B, HD, TN, lower_ref, h_ref, W20_ref, W21_ref, b2_ref,
               We_ref, be_ref, Wo_ref, bo_ref, out_ref, m_ref):
    i = pl.program_id(0)
    nsteps = pl.num_programs(0)
    L = lower_ref[...].astype(jnp.bfloat16)
    agg = jnp.dot(L, h_ref[...], preferred_element_type=jnp.float32)
    Hi = h_ref[pl.ds(i * TN, TN), :]
    A = agg.reshape(TN * B, HD).astype(jnp.bfloat16)
    Hf = Hi.reshape(TN * B, HD)
    G = (jnp.dot(Hf, W20_ref[...], preferred_element_type=jnp.float32)
         + jnp.dot(A, W21_ref[...], preferred_element_type=jnp.float32)
         + b2_ref[...])
    G = jnp.maximum(G, 0.0)
    Gm = jnp.max(G.reshape(TN, B * HD), axis=0, keepdims=True)

    @pl.when(i == 0)
    def _():
        m_ref[...] = Gm

    @pl.when(i > 0)
    def _():
        m_ref[...] = jnp.maximum(m_ref[...], Gm)

    @pl.when(i == nsteps - 1)
    def _():
        mm = m_ref[...].reshape(B, HD)
        t = jnp.dot(mm, We_ref[...], preferred_element_type=jnp.float32)
        t = jnp.maximum(t + be_ref[...], 0.0)
        out_ref[...] = (jnp.dot(t, Wo_ref[...],
                                preferred_element_type=jnp.float32)
                        + bo_ref[...])


def kernel(x, lower, _, W1_0, W1_1, b1, W2_0, W2_1, b2, We, be, Wo, bo):
    B, N, _d = x.shape
    HD = W1_0.shape[1]
    HFF = We.shape[1]
    NC = Wo.shape[1]

    xT = x[:, :, 0].T
    b1r = b1.reshape(1, HD)
    b2r = b2.reshape(1, HD)
    ber = be.reshape(1, HFF)
    bor = bo.reshape(1, NC)

    h2d = pl.pallas_call(
        functools.partial(_l1_kernel, B, TN1),
        grid=(N // TN1,),
        in_specs=[
            pl.BlockSpec((TN1, N), lambda i: (i, 0)),
            pl.BlockSpec((N, B), lambda i: (0, 0)),
            pl.BlockSpec((1, HD), lambda i: (0, 0)),
            pl.BlockSpec((1, HD), lambda i: (0, 0)),
            pl.BlockSpec((1, HD), lambda i: (0, 0)),
        ],
        out_specs=pl.BlockSpec((TN1, B * HD), lambda i: (i, 0)),
        out_shape=jax.ShapeDtypeStruct((N, B * HD), jnp.bfloat16),
    )(lower, xT, W1_0, W1_1, b1r)

    out = pl.pallas_call(
        functools.partial(_l2_kernel, B, HD, TN2),
        grid=(N // TN2,),
        in_specs=[
            pl.BlockSpec((TN2, N), lambda i: (i, 0)),
            pl.BlockSpec((N, B * HD), lambda i: (0, 0)),
            pl.BlockSpec((HD, HD), lambda i: (0, 0)),
            pl.BlockSpec((HD, HD), lambda i: (0, 0)),
            pl.BlockSpec((1, HD), lambda i: (0, 0)),
            pl.BlockSpec((HD, HFF), lambda i: (0, 0)),
            pl.BlockSpec((1, HFF), lambda i: (0, 0)),
            pl.BlockSpec((HFF, NC), lambda i: (0, 0)),
            pl.BlockSpec((1, NC), lambda i: (0, 0)),
        ],
        out_specs=pl.BlockSpec((B, NC), lambda i: (0, 0)),
        out_shape=jax.ShapeDtypeStruct((B, NC), jnp.float32),
        scratch_shapes=[pltpu.VMEM((1, B * HD), jnp.float32)],
    )(lower, h2d, W2_0.astype(jnp.bfloat16), W2_1.astype(jnp.bfloat16),
      b2r, We, ber, Wo, bor)

    return out


# speedup vs baseline: 1.4235x; 1.0489x over previous
import functools

import jax
import jax.numpy as jnp
from jax.experimental import pallas as pl
from jax.experimental.pallas import tpu as pltpu

TN = 256


def _fused_kernel(B, HD, nblk, lower_ref, xT_ref, W10_ref, W11_ref, b1_ref,
                  W20_ref, W21_ref, b2_ref, We_ref, be_ref, Wo_ref, bo_ref,
                  out_ref, L16_ref, h_ref, m_ref):
    i = pl.program_id(0)

    @pl.when(i < nblk)
    def _phase1():
        L16 = lower_ref[...].astype(jnp.bfloat16)
        L16_ref[pl.ds(i * TN, TN), :] = L16
        xT16 = xT_ref[...].astype(jnp.bfloat16)
        s = jnp.dot(L16, xT16, preferred_element_type=jnp.float32)
        xr = xT_ref[pl.ds(i * TN, TN), :]
        W10 = W10_ref[...]
        W11 = W11_ref[...]
        b1 = b1_ref[...]
        pieces = []
        for b in range(B):
            hb = xr[:, b:b + 1] * W10 + s[:, b:b + 1] * W11 + b1
            pieces.append(jnp.maximum(hb, 0.0))
        h_ref[pl.ds(i * TN, TN), :] = (
            jnp.concatenate(pieces, axis=1).astype(jnp.bfloat16))

    @pl.when(i >= nblk)
    def _phase2():
        j = i - nblk
        Lb = L16_ref[pl.ds(j * TN, TN), :]
        agg = jnp.dot(Lb, h_ref[...],
                      preferred_element_type=jnp.float32)
        Hi = h_ref[pl.ds(j * TN, TN), :]
        A = agg.reshape(TN * B, HD).astype(jnp.bfloat16)
        Hf = Hi.reshape(TN * B, HD)
        G = (jnp.dot(Hf, W20_ref[...], preferred_element_type=jnp.float32)
             + jnp.dot(A, W21_ref[...], preferred_element_type=jnp.float32)
             + b2_ref[...])
        G = jnp.maximum(G, 0.0)
        Gm = jnp.max(G.reshape(TN, B * HD), axis=0, keepdims=True)

        @pl.when(j == 0)
        def _():
            m_ref[...] = Gm

        @pl.when(j > 0)
        def _():
            m_ref[...] = jnp.maximum(m_ref[...], Gm)

        @pl.when(j == nblk - 1)
        def _():
            mm = m_ref[...].reshape(B, HD)
            t = jnp.dot(mm, We_ref[...], preferred_element_type=jnp.float32)
            t = jnp.maximum(t + be_ref[...], 0.0)
            out_ref[...] = (jnp.dot(t, Wo_ref[...],
                                    preferred_element_type=jnp.float32)
                            + bo_ref[...])


def kernel(x, lower, _, W1_0, W1_1, b1, W2_0, W2_1, b2, We, be, Wo, bo):
    B, N, _d = x.shape
    HD = W1_0.shape[1]
    HFF = We.shape[1]
    NC = Wo.shape[1]
    nblk = N // TN

    xT = x[:, :, 0].T
    b1r = b1.reshape(1, HD)
    b2r = b2.reshape(1, HD)
    ber = be.reshape(1, HFF)
    bor = bo.reshape(1, NC)

    cidx = lambda i: (0, 0)
    out = pl.pallas_call(
        functools.partial(_fused_kernel, B, HD, nblk),
        grid=(2 * nblk,),
        in_specs=[
            pl.BlockSpec((TN, N), lambda i: (jnp.minimum(i, nblk - 1), 0)),
            pl.BlockSpec((N, B), cidx),
            pl.BlockSpec((1, HD), cidx),
            pl.BlockSpec((1, HD), cidx),
            pl.BlockSpec((1, HD), cidx),
            pl.BlockSpec((HD, HD), cidx),
            pl.BlockSpec((HD, HD), cidx),
            pl.BlockSpec((1, HD), cidx),
            pl.BlockSpec((HD, HFF), cidx),
            pl.BlockSpec((1, HFF), cidx),
            pl.BlockSpec((HFF, NC), cidx),
            pl.BlockSpec((1, NC), cidx),
        ],
        out_specs=pl.BlockSpec((B, NC), cidx),
        out_shape=jax.ShapeDtypeStruct((B, NC), jnp.float32),
        scratch_shapes=[
            pltpu.VMEM((N, N), jnp.bfloat16),
            pltpu.VMEM((N, B * HD), jnp.bfloat16),
            pltpu.VMEM((1, B * HD), jnp.float32),
        ],
    )(lower, xT, W1_0, W1_1, b1r,
      W2_0.astype(jnp.bfloat16), W2_1.astype(jnp.bfloat16), b2r,
      We, ber, Wo, bor)

    return out


# speedup vs baseline: 1.5613x; 1.0968x over previous
import functools

import jax
import jax.numpy as jnp
from jax import lax
from jax.experimental import pallas as pl
from jax.experimental.pallas import tpu as pltpu

TN = 256


def _fused_kernel(B, HD, nblk, lower_ref, x_ref, W10_ref, W11_ref, b1_ref,
                  W20_ref, W21_ref, b2_ref, We_ref, be_ref, Wo_ref, bo_ref,
                  out_ref, h_ref, m_ref):
    i = pl.program_id(0)

    @pl.when(i < nblk)
    def _phase1():
        L = lower_ref[...]
        sT = lax.dot_general(x_ref[...], L, (((1,), (1,)), ((), ())),
                             preferred_element_type=jnp.float32)
        s = sT.T
        xr = x_ref[:, pl.ds(i * TN, TN)].T
        W10 = W10_ref[...]
        W11 = W11_ref[...]
        b1 = b1_ref[...]
        pieces = []
        for b in range(B):
            hb = xr[:, b:b + 1] * W10 + s[:, b:b + 1] * W11 + b1
            pieces.append(jnp.maximum(hb, 0.0))
        h_ref[pl.ds(i * TN, TN), :] = jnp.concatenate(pieces, axis=1)

    @pl.when(i >= nblk)
    def _phase2():
        j = i - nblk
        L = lower_ref[...]
        agg = jnp.dot(L, h_ref[...],
                      preferred_element_type=jnp.float32)
        Hi = h_ref[pl.ds(j * TN, TN), :]
        A = agg.reshape(TN * B, HD)
        Hf = Hi.reshape(TN * B, HD)
        G = (jnp.dot(Hf, W20_ref[...], preferred_element_type=jnp.float32)
             + jnp.dot(A, W21_ref[...], preferred_element_type=jnp.float32)
             + b2_ref[...])
        G = jnp.maximum(G, 0.0)
        Gm = jnp.max(G.reshape(TN, B * HD), axis=0, keepdims=True)

        @pl.when(j == 0)
        def _():
            m_ref[...] = Gm

        @pl.when(j > 0)
        def _():
            m_ref[...] = jnp.maximum(m_ref[...], Gm)

        @pl.when(j == nblk - 1)
        def _():
            mm = m_ref[...].reshape(B, HD)
            t = jnp.dot(mm, We_ref[...], preferred_element_type=jnp.float32)
            t = jnp.maximum(t + be_ref[...], 0.0)
            out_ref[...] = (jnp.dot(t, Wo_ref[...],
                                    preferred_element_type=jnp.float32)
                            + bo_ref[...])


def kernel(x, lower, _, W1_0, W1_1, b1, W2_0, W2_1, b2, We, be, Wo, bo):
    B, N, _d = x.shape
    HD = W1_0.shape[1]
    HFF = We.shape[1]
    NC = Wo.shape[1]
    nblk = N // TN

    x2d = x[:, :, 0]
    b1r = b1.reshape(1, HD)
    b2r = b2.reshape(1, HD)
    ber = be.reshape(1, HFF)
    bor = bo.reshape(1, NC)

    cidx = lambda i: (0, 0)
    out = pl.pallas_call(
        functools.partial(_fused_kernel, B, HD, nblk),
        grid=(2 * nblk,),
        in_specs=[
            pl.BlockSpec((TN, N),
                         lambda i: (jnp.where(i < nblk, i, i - nblk), 0)),
            pl.BlockSpec((B, N), cidx),
            pl.BlockSpec((1, HD), cidx),
            pl.BlockSpec((1, HD), cidx),
            pl.BlockSpec((1, HD), cidx),
            pl.BlockSpec((HD, HD), cidx),
            pl.BlockSpec((HD, HD), cidx),
            pl.BlockSpec((1, HD), cidx),
            pl.BlockSpec((HD, HFF), cidx),
            pl.BlockSpec((1, HFF), cidx),
            pl.BlockSpec((HFF, NC), cidx),
            pl.BlockSpec((1, NC), cidx),
        ],
        out_specs=pl.BlockSpec((B, NC), cidx),
        out_shape=jax.ShapeDtypeStruct((B, NC), jnp.float32),
        scratch_shapes=[
            pltpu.VMEM((N, B * HD), jnp.float32),
            pltpu.VMEM((1, B * HD), jnp.float32),
        ],
    )(lower, x2d, W1_0, W1_1, b1r, W2_0, W2_1, b2r, We, ber, Wo, bor)

    return out
